# baseline (speedup 1.0000x reference)
import functools

import jax
import jax.numpy as jnp
from jax import lax
from jax.experimental import pallas as pl
from jax.experimental.pallas import tpu as pltpu
from jax.experimental.pallas import tpu_sc as plsc

_EPS = 1e-5

_NC = 2
_NS = 16
_NW = _NC * _NS
_CH = 128
_CHG = 128
_SLABS = 6


def _ln(h, g, b):
    mu = jnp.mean(h, axis=-1, keepdims=True)
    var = jnp.mean((h - mu) ** 2, axis=-1, keepdims=True)
    return (h - mu) * lax.rsqrt(var + _EPS) * g + b


def _precomp_body(x_ref, w1a_ref, w1b_ref, b1_ref, p_ref, q_ref):
    xb = x_ref[...]
    p_ref[...] = (
        jnp.dot(xb, w1a_ref[...], preferred_element_type=jnp.float32) + b1_ref[...]
    )
    q_ref[...] = jnp.dot(xb, w1b_ref[...], preferred_element_type=jnp.float32)


def _edge_mlp_body(g_ref, a_ref, w1c_ref, g1_ref, be1_ref, w2_ref, b2_ref, o_ref):
    t = lax.dot_general(
        a_ref[...], w1c_ref[...], (((0,), (0,)), ((), ())),
        preferred_element_type=jnp.float32,
    )
    h = g_ref[...] + t
    h = jnp.maximum(_ln(h, g1_ref[...], be1_ref[...]), 0.0)
    m = (
        jnp.dot(
            h.astype(jnp.bfloat16), w2_ref[...], preferred_element_type=jnp.float32
        )
        + b2_ref[...]
    )
    o_ref[...] = jnp.maximum(m, 0.0)


def _update_body(
    x_ref, a0_ref, a1_ref, w3a_ref, w3b_ref, b3_ref, g3_ref, be3_ref, o_ref
):
    xb = x_ref[...]
    ag = a0_ref[...] + a1_ref[...]
    u = (
        jnp.dot(xb, w3a_ref[...], preferred_element_type=jnp.float32)
        + jnp.dot(ag, w3b_ref[...], preferred_element_type=jnp.float32)
        + b3_ref[...]
    )
    u = jnp.maximum(_ln(u, g3_ref[...], be3_ref[...]), 0.0)
    o_ref[...] = xb + u


def _make_warmup():
    mesh = plsc.VectorSubcoreMesh(core_axis_name="c", subcore_axis_name="s")

    @functools.partial(
        pl.kernel,
        mesh=mesh,
        out_type=jax.ShapeDtypeStruct((_NW, 128), jnp.int32),
        scratch_types=[pltpu.VMEM((128,), jnp.int32)],
    )
    def warm_k(in_hbm, out_hbm, buf):
        cid = lax.axis_index("c")
        sid = lax.axis_index("s")
        wid = sid * _NC + cid
        pltpu.sync_copy(in_hbm.at[wid], buf)
        pltpu.sync_copy(buf, out_hbm.at[wid])

    return warm_k


def _make_gather(d, e_pad):
    nch = e_pad // (_NW * _CHG)
    epw = nch * _CHG
    mesh = plsc.VectorSubcoreMesh(core_axis_name="c", subcore_axis_name="s")

    assert nch >= 4

    @functools.partial(
        pl.kernel,
        mesh=mesh,
        out_type=jax.ShapeDtypeStruct((e_pad, d), jnp.float32),
        scratch_types=[
            pltpu.VMEM((nch, _CHG), jnp.int32),
            pltpu.VMEM((nch, _CHG), jnp.int32),
            pltpu.VMEM((2, _CHG, d), jnp.float32),
            pltpu.VMEM((2, _CHG, d), jnp.float32),
            pltpu.SemaphoreType.DMA,
            pltpu.SemaphoreType.DMA,
            pltpu.SemaphoreType.DMA,
            pltpu.SemaphoreType.DMA,
        ],
    )
    def gather_k(
        p_hbm, q_hbm, row2_hbm, col2_hbm, out_hbm, ridx, cidx, bufp, bufq,
        sg0, sg1, sw0, sw1
    ):
        cid = lax.axis_index("c")
        sid = lax.axis_index("s")
        wid = sid * _NC + cid
        pltpu.sync_copy(row2_hbm.at[wid], ridx)
        pltpu.sync_copy(col2_hbm.at[wid], cidx)
        sg = (sg0, sg1)
        sw = (sw0, sw1)

        def issue(j, s):
            pltpu.async_copy(p_hbm.at[ridx.at[j]], bufp.at[s], sg[s])
            pltpu.async_copy(q_hbm.at[cidx.at[j]], bufq.at[s], sg[s])

        def wait_gather(s):
            pltpu.make_async_copy(p_hbm.at[ridx.at[0]], bufp.at[s], sg[s]).wait()
            pltpu.make_async_copy(q_hbm.at[cidx.at[0]], bufq.at[s], sg[s]).wait()

        def wait_write(s):
            pltpu.make_async_copy(
                bufp.at[s], out_hbm.at[pl.ds(wid * epw, _CHG)], sw[s]
            ).wait()

        def process(j, s):
            wait_gather(s)

            def addrow(r, c2):
                for k in range(d // 16):
                    v = bufq[s, r, pl.ds(k * 16, 16)]
                    plsc.addupdate(bufp.at[s, r, pl.ds(k * 16, 16)], v)
                return c2

            lax.fori_loop(0, _CHG, addrow, 0)
            pltpu.async_copy(
                bufp.at[s], out_hbm.at[pl.ds(wid * epw + j * _CHG, _CHG)], sw[s]
            )

        npairs = (nch - 2) // 2
        ntail = nch - 2 * npairs
        issue(0, 0)
        issue(1, 1)

        def pair(jj, carry):
            j = 2 * jj
            process(j, 0)
            wait_write(0)
            issue(j + 2, 0)
            process(j + 1, 1)
            wait_write(1)
            issue(j + 3, 1)
            return carry

        lax.fori_loop(0, npairs, pair, 0)
        t0 = 2 * npairs
        if ntail == 3:
            process(t0, 0)
            wait_write(0)
            issue(nch - 1, 0)
            process(t0 + 1, 1)
            process(nch - 1, 0)
        else:
            process(t0, 0)
            process(t0 + 1, 1)
        wait_write(0)
        wait_write(1)

    return gather_k


def _make_scatter(n_acc, d, slab_nchs, ntr, init):
    nslab = len(slab_nchs)
    nch_max = max(slab_nchs)
    mesh = plsc.VectorSubcoreMesh(core_axis_name="c", subcore_axis_name="s")

    assert min(slab_nchs) >= 4

    @functools.partial(
        pl.kernel,
        mesh=mesh,
        out_type=jax.ShapeDtypeStruct((_NC, n_acc, d), jnp.float32),
        scratch_types=[
            pltpu.VMEM((nch_max, _CH), jnp.int32),
            pltpu.VMEM((2, _CH, d), jnp.float32),
            pltpu.VMEM_SHARED((n_acc, d), jnp.float32),
            pltpu.SemaphoreType.DMA,
            pltpu.SemaphoreType.DMA,
        ],
    )
    def scatter_k(*refs):
        m_hbms = refs[0:nslab]
        r_hbms = refs[nslab : 2 * nslab]
        nin = 2 * nslab + (0 if init else 1)
        out_hbm = refs[nin]
        idx, buf, acc, sr0, sr1 = refs[nin + 1 :]
        sr = (sr0, sr1)
        cid = lax.axis_index("c")
        sid = lax.axis_index("s")
        wid = sid * _NC + cid
        z = jnp.zeros((16,), jnp.float32)

        if init:
            def zrow(i, carry):
                for k in range(d // 16):
                    buf[0, i, pl.ds(k * 16, 16)] = z
                return carry

            lax.fori_loop(0, _CH, zrow, 0)
            for off in range(0, ntr, _CH):
                sz = min(_CH, ntr - off)
                pltpu.sync_copy(
                    buf.at[0, pl.ds(0, sz)], acc.at[pl.ds(sid * ntr + off, sz)]
                )
        else:
            acc_in = refs[2 * nslab]
            for off in range(0, ntr, _CH):
                sz = min(_CH, ntr - off)
                pltpu.sync_copy(
                    acc_in.at[cid, pl.ds(sid * ntr + off, sz)],
                    buf.at[0, pl.ds(0, sz)],
                )
                pltpu.sync_copy(
                    buf.at[0, pl.ds(0, sz)], acc.at[pl.ds(sid * ntr + off, sz)]
                )
        plsc.subcore_barrier()

        for sb in range(nslab):
            m_hbm = m_hbms[sb]
            nch = slab_nchs[sb]
            epw = nch * _CH
            pltpu.sync_copy(r_hbms[sb].at[wid], idx.at[pl.ds(0, nch)])

            def issue(j, s):
                pltpu.async_copy(
                    m_hbm.at[pl.ds(wid * epw + j * _CH, _CH)], buf.at[s], sr[s]
                )

            def wait_read(s):
                pltpu.make_async_copy(
                    m_hbm.at[pl.ds(wid * epw, _CH)], buf.at[s], sr[s]
                ).wait()

            def process(j, s):
                wait_read(s)
                pltpu.sync_copy(buf.at[s], acc.at[idx.at[j]], add=True)

            npairs = (nch - 2) // 2
            ntail = nch - 2 * npairs
            issue(0, 0)
            issue(1, 1)

            def pair(jj, carry):
                j = 2 * jj
                process(j, 0)
                issue(j + 2, 0)
                process(j + 1, 1)
                issue(j + 3, 1)
                return carry

            lax.fori_loop(0, npairs, pair, 0)
            t0 = 2 * npairs
            if ntail == 3:
                process(t0, 0)
                issue(nch - 1, 0)
                process(t0 + 1, 1)
                process(nch - 1, 0)
            else:
                process(t0, 0)
                process(t0 + 1, 1)

        plsc.subcore_barrier()
        pltpu.sync_copy(
            acc.at[pl.ds(sid * ntr, ntr)], out_hbm.at[cid, pl.ds(sid * ntr, ntr)]
        )

    return scatter_k


def kernel(x, edge_index, edge_attr, W1, b1, g1, beta1, W2, b2, W3, b3, g3, beta3):
    n, d = x.shape
    e = edge_attr.shape[0]
    assert d % 16 == 0

    grain = _NW * _CH
    e_pad = -(-e // grain) * grain
    units = e_pad // grain
    fr = (0.05, 0.10, 0.20, 0.25, 0.25, 0.15)
    slab_units = [max(4, int(round(units * f))) for f in fr[:_SLABS]]
    slab_units[3] += units - sum(slab_units)

    ntr = -(-(n + 1) // _NS)
    ntr = -(-ntr // 8) * 8
    n_acc = ntr * _NS

    warm = _make_warmup()(jnp.zeros((_NW, 128), jnp.int32))
    wz = warm[0, 0]

    row = edge_index[0] + wz
    col = edge_index[1] + wz
    padg = jnp.zeros((e_pad - e,), jnp.int32)
    row_pg = jnp.concatenate([row, padg])
    col_pg = jnp.concatenate([col, padg])
    row_ps = jnp.concatenate([row, jnp.full((e_pad - e,), n, jnp.int32)])
    ea_t = jnp.pad(edge_attr.T, ((0, 0), (0, e_pad - e)))

    w1a = W1[:d]
    w1b = W1[d : 2 * d]
    w1c = W1[2 * d :]
    w3a = W3[:d]
    w3b = W3[d:]
    b1r = b1.reshape(1, d)
    g1r = g1.reshape(1, d)
    be1r = beta1.reshape(1, d)
    b2r = b2.reshape(1, d)
    b3r = b3.reshape(1, d)
    g3r = g3.reshape(1, d)
    be3r = beta3.reshape(1, d)

    bn = 1000 if n % 1000 == 0 else 8
    assert n % bn == 0
    full = pl.BlockSpec((d, d), lambda i: (0, 0))
    vec = pl.BlockSpec((1, d), lambda i: (0, 0))
    x_pad = jnp.pad(x, ((0, n_acc - n), (0, 0)))
    p_arr, q_arr = pl.pallas_call(
        _precomp_body,
        grid=(_NS,),
        in_specs=[pl.BlockSpec((ntr, d), lambda i: (i, 0)), full, full, vec],
        out_specs=[
            pl.BlockSpec((ntr, d), lambda i: (i, 0)),
            pl.BlockSpec((ntr, d), lambda i: (i, 0)),
        ],
        out_shape=[
            jax.ShapeDtypeStruct((n_acc, d), jnp.float32),
            jax.ShapeDtypeStruct((n_acc, d), jnp.float32),
        ],
    )(x_pad, w1a, w1b, b1r)

    be = 1024
    w2bf = W2.astype(jnp.bfloat16)
    gmakers = {}
    m_slabs = []
    rs_slabs = []
    offs = 0
    for s in range(_SLABS):
        e_sl = slab_units[s] * grain
        sl = slice(offs, offs + e_sl)
        offs += e_sl
        nch_g = e_sl // (_NW * _CHG)
        rg = row_pg[sl].reshape(_NW, nch_g, _CHG)
        cg = col_pg[sl].reshape(_NW, nch_g, _CHG)
        rs_slabs.append(row_ps[sl].reshape(_NW, slab_units[s], _CH))
        if e_sl not in gmakers:
            gmakers[e_sl] = _make_gather(d, e_sl)
        g_s = gmakers[e_sl](p_arr, q_arr, rg, cg)
        m_s = pl.pallas_call(
            _edge_mlp_body,
            grid=(e_sl // be,),
            in_specs=[
                pl.BlockSpec((be, d), lambda i: (i, 0)),
                pl.BlockSpec((2, be), lambda i: (0, i)),
                pl.BlockSpec((2, d), lambda i: (0, 0)),
                vec,
                vec,
                full,
                vec,
            ],
            out_specs=pl.BlockSpec((be, d), lambda i: (i, 0)),
            out_shape=jax.ShapeDtypeStruct((e_sl, d), jnp.float32),
        )(g_s, ea_t[:, sl], w1c, g1r, be1r, w2bf, b2r)
        m_slabs.append(m_s)

    ksp = _SLABS - 1
    parts_a = _make_scatter(n_acc, d, tuple(slab_units[:ksp]), ntr, True)(
        *m_slabs[:ksp], *rs_slabs[:ksp]
    )
    parts = _make_scatter(n_acc, d, tuple(slab_units[ksp:]), ntr, False)(
        *m_slabs[ksp:], *rs_slabs[ksp:], parts_a
    )
    a0 = parts[0, :n]
    a1 = parts[1, :n]

    out = pl.pallas_call(
        _update_body,
        grid=(n // bn,),
        in_specs=[
            pl.BlockSpec((bn, d), lambda i: (i, 0)),
            pl.BlockSpec((bn, d), lambda i: (i, 0)),
            pl.BlockSpec((bn, d), lambda i: (i, 0)),
            full,
            full,
            vec,
            vec,
            vec,
        ],
        out_specs=pl.BlockSpec((bn, d), lambda i: (i, 0)),
        out_shape=jax.ShapeDtypeStruct((n, d), jnp.float32),
    )(x, a0, a1, w3a, w3b, b3r, g3r, be3r)
    return out


# speedup vs baseline: 1.0302x; 1.0302x over previous
import functools

import jax
import jax.numpy as jnp
from jax import lax
from jax.experimental import pallas as pl
from jax.experimental.pallas import tpu as pltpu
from jax.experimental.pallas import tpu_sc as plsc

_EPS = 1e-5

_NC = 2
_NS = 16
_NW = _NC * _NS
_CH = 128
_CHG = 128
_SLABS = 6


def _ln(h, g, b):
    mu = jnp.mean(h, axis=-1, keepdims=True)
    var = jnp.mean((h - mu) ** 2, axis=-1, keepdims=True)
    return (h - mu) * lax.rsqrt(var + _EPS) * g + b


def _precomp_body(x_ref, w1a_ref, w1b_ref, b1_ref, p_ref, q_ref):
    xb = x_ref[...]
    p_ref[...] = (
        jnp.dot(xb, w1a_ref[...], preferred_element_type=jnp.float32) + b1_ref[...]
    )
    q_ref[...] = jnp.dot(xb, w1b_ref[...], preferred_element_type=jnp.float32)


def _edge_mlp_body(g_ref, a_ref, w1c_ref, g1_ref, be1_ref, w2_ref, b2_ref, o_ref):
    t = lax.dot_general(
        a_ref[...], w1c_ref[...], (((0,), (0,)), ((), ())),
        preferred_element_type=jnp.float32,
    )
    h = g_ref[...] + t
    h = jnp.maximum(_ln(h, g1_ref[...], be1_ref[...]), 0.0)
    m = (
        jnp.dot(
            h.astype(jnp.bfloat16), w2_ref[...], preferred_element_type=jnp.float32
        )
        + b2_ref[...]
    )
    o_ref[...] = jnp.maximum(m, 0.0)


def _update_body(
    x_ref, a0_ref, a1_ref, w3a_ref, w3b_ref, b3_ref, g3_ref, be3_ref, o_ref
):
    xb = x_ref[...]
    ag = a0_ref[...] + a1_ref[...]
    u = (
        jnp.dot(xb, w3a_ref[...], preferred_element_type=jnp.float32)
        + jnp.dot(ag, w3b_ref[...], preferred_element_type=jnp.float32)
        + b3_ref[...]
    )
    u = jnp.maximum(_ln(u, g3_ref[...], be3_ref[...]), 0.0)
    o_ref[...] = xb + u


def _make_warmup():
    mesh = plsc.VectorSubcoreMesh(core_axis_name="c", subcore_axis_name="s")

    @functools.partial(
        pl.kernel,
        mesh=mesh,
        out_type=jax.ShapeDtypeStruct((_NW, 128), jnp.int32),
        scratch_types=[pltpu.VMEM((128,), jnp.int32)],
    )
    def warm_k(in_hbm, out_hbm, buf):
        cid = lax.axis_index("c")
        sid = lax.axis_index("s")
        wid = sid * _NC + cid
        pltpu.sync_copy(in_hbm.at[wid], buf)
        pltpu.sync_copy(buf, out_hbm.at[wid])

    return warm_k


def _make_gather(d, e_pad):
    nch = e_pad // (_NW * _CHG)
    epw = nch * _CHG
    mesh = plsc.VectorSubcoreMesh(core_axis_name="c", subcore_axis_name="s")

    assert nch >= 4

    @functools.partial(
        pl.kernel,
        mesh=mesh,
        out_type=jax.ShapeDtypeStruct((e_pad, d), jnp.float32),
        scratch_types=[
            pltpu.VMEM((nch, _CHG), jnp.int32),
            pltpu.VMEM((nch, _CHG), jnp.int32),
            pltpu.VMEM((2, _CHG, d), jnp.float32),
            pltpu.VMEM((2, _CHG, d), jnp.float32),
            pltpu.SemaphoreType.DMA,
            pltpu.SemaphoreType.DMA,
            pltpu.SemaphoreType.DMA,
            pltpu.SemaphoreType.DMA,
        ],
    )
    def gather_k(
        p_hbm, q_hbm, row2_hbm, col2_hbm, out_hbm, ridx, cidx, bufp, bufq,
        sg0, sg1, sw0, sw1
    ):
        cid = lax.axis_index("c")
        sid = lax.axis_index("s")
        wid = sid * _NC + cid
        pltpu.sync_copy(row2_hbm.at[wid], ridx)
        pltpu.sync_copy(col2_hbm.at[wid], cidx)
        sg = (sg0, sg1)
        sw = (sw0, sw1)

        def issue(j, s):
            pltpu.async_copy(p_hbm.at[ridx.at[j]], bufp.at[s], sg[s])
            pltpu.async_copy(q_hbm.at[cidx.at[j]], bufq.at[s], sg[s])

        def wait_gather(s):
            pltpu.make_async_copy(p_hbm.at[ridx.at[0]], bufp.at[s], sg[s]).wait()
            pltpu.make_async_copy(q_hbm.at[cidx.at[0]], bufq.at[s], sg[s]).wait()

        def wait_write(s):
            pltpu.make_async_copy(
                bufp.at[s], out_hbm.at[pl.ds(wid * epw, _CHG)], sw[s]
            ).wait()

        def process(j, s):
            wait_gather(s)

            def addrow(r, c2):
                for k in range(d // 16):
                    v = bufq[s, r, pl.ds(k * 16, 16)]
                    plsc.addupdate(bufp.at[s, r, pl.ds(k * 16, 16)], v)
                return c2

            lax.fori_loop(0, _CHG, addrow, 0)
            pltpu.async_copy(
                bufp.at[s], out_hbm.at[pl.ds(wid * epw + j * _CHG, _CHG)], sw[s]
            )

        npairs = (nch - 2) // 2
        ntail = nch - 2 * npairs
        issue(0, 0)
        issue(1, 1)

        def pair(jj, carry):
            j = 2 * jj
            process(j, 0)
            wait_write(0)
            issue(j + 2, 0)
            process(j + 1, 1)
            wait_write(1)
            issue(j + 3, 1)
            return carry

        lax.fori_loop(0, npairs, pair, 0)
        t0 = 2 * npairs
        if ntail == 3:
            process(t0, 0)
            wait_write(0)
            issue(nch - 1, 0)
            process(t0 + 1, 1)
            process(nch - 1, 0)
        else:
            process(t0, 0)
            process(t0 + 1, 1)
        wait_write(0)
        wait_write(1)

    return gather_k


def _make_scatter(n_acc, d, slab_nchs, ntr, init):
    nslab = len(slab_nchs)
    nch_max = max(slab_nchs)
    mesh = plsc.VectorSubcoreMesh(core_axis_name="c", subcore_axis_name="s")

    assert min(slab_nchs) >= 4

    @functools.partial(
        pl.kernel,
        mesh=mesh,
        out_type=jax.ShapeDtypeStruct((_NC, n_acc, d), jnp.float32),
        scratch_types=[
            pltpu.VMEM((nch_max, _CH), jnp.int32),
            pltpu.VMEM((2, _CH, d), jnp.float32),
            pltpu.VMEM_SHARED((n_acc, d), jnp.float32),
            pltpu.SemaphoreType.DMA,
            pltpu.SemaphoreType.DMA,
        ],
    )
    def scatter_k(*refs):
        m_hbms = refs[0:nslab]
        r_hbms = refs[nslab : 2 * nslab]
        nin = 2 * nslab + (0 if init else 1)
        out_hbm = refs[nin]
        idx, buf, acc, sr0, sr1 = refs[nin + 1 :]
        sr = (sr0, sr1)
        cid = lax.axis_index("c")
        sid = lax.axis_index("s")
        wid = sid * _NC + cid
        z = jnp.zeros((16,), jnp.float32)

        if init:
            def zrow(i, carry):
                for k in range(d // 16):
                    buf[0, i, pl.ds(k * 16, 16)] = z
                return carry

            lax.fori_loop(0, _CH, zrow, 0)
            for off in range(0, ntr, _CH):
                sz = min(_CH, ntr - off)
                pltpu.sync_copy(
                    buf.at[0, pl.ds(0, sz)], acc.at[pl.ds(sid * ntr + off, sz)]
                )
        else:
            acc_in = refs[2 * nslab]
            for off in range(0, ntr, _CH):
                sz = min(_CH, ntr - off)
                pltpu.sync_copy(
                    acc_in.at[cid, pl.ds(sid * ntr + off, sz)],
                    buf.at[0, pl.ds(0, sz)],
                )
                pltpu.sync_copy(
                    buf.at[0, pl.ds(0, sz)], acc.at[pl.ds(sid * ntr + off, sz)]
                )
        plsc.subcore_barrier()

        for sb in range(nslab):
            m_hbm = m_hbms[sb]
            nch = slab_nchs[sb]
            epw = nch * _CH
            pltpu.sync_copy(r_hbms[sb].at[wid], idx.at[pl.ds(0, nch)])

            def issue(j, s):
                pltpu.async_copy(
                    m_hbm.at[pl.ds(wid * epw + j * _CH, _CH)], buf.at[s], sr[s]
                )

            def wait_read(s):
                pltpu.make_async_copy(
                    m_hbm.at[pl.ds(wid * epw, _CH)], buf.at[s], sr[s]
                ).wait()

            def process(j, s):
                wait_read(s)
                pltpu.sync_copy(buf.at[s], acc.at[idx.at[j]], add=True)

            npairs = (nch - 2) // 2
            ntail = nch - 2 * npairs
            issue(0, 0)
            issue(1, 1)

            def pair(jj, carry):
                j = 2 * jj
                process(j, 0)
                issue(j + 2, 0)
                process(j + 1, 1)
                issue(j + 3, 1)
                return carry

            lax.fori_loop(0, npairs, pair, 0)
            t0 = 2 * npairs
            if ntail == 3:
                process(t0, 0)
                issue(nch - 1, 0)
                process(t0 + 1, 1)
                process(nch - 1, 0)
            else:
                process(t0, 0)
                process(t0 + 1, 1)

        plsc.subcore_barrier()
        pltpu.sync_copy(
            acc.at[pl.ds(sid * ntr, ntr)], out_hbm.at[cid, pl.ds(sid * ntr, ntr)]
        )

    return scatter_k


def kernel(x, edge_index, edge_attr, W1, b1, g1, beta1, W2, b2, W3, b3, g3, beta3):
    n, d = x.shape
    e = edge_attr.shape[0]
    assert d % 16 == 0

    grain = _NW * _CH
    e_pad = -(-e // grain) * grain
    units = e_pad // grain
    fr = (0.05, 0.24, 0.24, 0.19, 0.15, 0.13)
    slab_units = [max(4, int(round(units * f))) for f in fr[:_SLABS]]
    slab_units[1] += units - sum(slab_units)

    ntr = -(-(n + 1) // _NS)
    ntr = -(-ntr // 8) * 8
    n_acc = ntr * _NS

    warm = _make_warmup()(jnp.zeros((_NW, 128), jnp.int32))
    wz = warm[0, 0]

    row = edge_index[0] + wz
    col = edge_index[1] + wz
    padg = jnp.zeros((e_pad - e,), jnp.int32)
    row_pg = jnp.concatenate([row, padg])
    col_pg = jnp.concatenate([col, padg])
    row_ps = jnp.concatenate([row, jnp.full((e_pad - e,), n, jnp.int32)])
    ea_t = jnp.pad(edge_attr.T, ((0, 0), (0, e_pad - e)))

    w1a = W1[:d]
    w1b = W1[d : 2 * d]
    w1c = W1[2 * d :]
    w3a = W3[:d]
    w3b = W3[d:]
    b1r = b1.reshape(1, d)
    g1r = g1.reshape(1, d)
    be1r = beta1.reshape(1, d)
    b2r = b2.reshape(1, d)
    b3r = b3.reshape(1, d)
    g3r = g3.reshape(1, d)
    be3r = beta3.reshape(1, d)

    bn = 1000 if n % 1000 == 0 else 8
    assert n % bn == 0
    full = pl.BlockSpec((d, d), lambda i: (0, 0))
    vec = pl.BlockSpec((1, d), lambda i: (0, 0))
    x_pad = jnp.pad(x, ((0, n_acc - n), (0, 0)))
    p_arr, q_arr = pl.pallas_call(
        _precomp_body,
        grid=(_NS,),
        in_specs=[pl.BlockSpec((ntr, d), lambda i: (i, 0)), full, full, vec],
        out_specs=[
            pl.BlockSpec((ntr, d), lambda i: (i, 0)),
            pl.BlockSpec((ntr, d), lambda i: (i, 0)),
        ],
        out_shape=[
            jax.ShapeDtypeStruct((n_acc, d), jnp.float32),
            jax.ShapeDtypeStruct((n_acc, d), jnp.float32),
        ],
    )(x_pad, w1a, w1b, b1r)

    be = 1024
    w2bf = W2.astype(jnp.bfloat16)
    gmakers = {}
    m_slabs = []
    rs_slabs = []
    offs = 0
    for s in range(_SLABS):
        e_sl = slab_units[s] * grain
        sl = slice(offs, offs + e_sl)
        offs += e_sl
        nch_g = e_sl // (_NW * _CHG)
        rg = row_pg[sl].reshape(_NW, nch_g, _CHG)
        cg = col_pg[sl].reshape(_NW, nch_g, _CHG)
        rs_slabs.append(row_ps[sl].reshape(_NW, slab_units[s], _CH))
        if e_sl not in gmakers:
            gmakers[e_sl] = _make_gather(d, e_sl)
        g_s = gmakers[e_sl](p_arr, q_arr, rg, cg)
        m_s = pl.pallas_call(
            _edge_mlp_body,
            grid=(e_sl // be,),
            in_specs=[
                pl.BlockSpec((be, d), lambda i: (i, 0)),
                pl.BlockSpec((2, be), lambda i: (0, i)),
                pl.BlockSpec((2, d), lambda i: (0, 0)),
                vec,
                vec,
                full,
                vec,
            ],
            out_specs=pl.BlockSpec((be, d), lambda i: (i, 0)),
            out_shape=jax.ShapeDtypeStruct((e_sl, d), jnp.float32),
        )(g_s, ea_t[:, sl], w1c, g1r, be1r, w2bf, b2r)
        m_slabs.append(m_s)

    ksp = _SLABS - 2
    parts_a = _make_scatter(n_acc, d, tuple(slab_units[:ksp]), ntr, True)(
        *m_slabs[:ksp], *rs_slabs[:ksp]
    )
    parts = _make_scatter(n_acc, d, tuple(slab_units[ksp:]), ntr, False)(
        *m_slabs[ksp:], *rs_slabs[ksp:], parts_a
    )
    a0 = parts[0, :n]
    a1 = parts[1, :n]

    out = pl.pallas_call(
        _update_body,
        grid=(n // bn,),
        in_specs=[
            pl.BlockSpec((bn, d), lambda i: (i, 0)),
            pl.BlockSpec((bn, d), lambda i: (i, 0)),
            pl.BlockSpec((bn, d), lambda i: (i, 0)),
            full,
            full,
            vec,
            vec,
            vec,
        ],
        out_specs=pl.BlockSpec((bn, d), lambda i: (i, 0)),
        out_shape=jax.ShapeDtypeStruct((n, d), jnp.float32),
    )(x, a0, a1, w3a, w3b, b3r, g3r, be3r)
    return out


# speedup vs baseline: 1.0361x; 1.0057x over previous
import functools

import jax
import jax.numpy as jnp
from jax import lax
from jax.experimental import pallas as pl
from jax.experimental.pallas import tpu as pltpu
from jax.experimental.pallas import tpu_sc as plsc

_EPS = 1e-5

_NC = 2
_NS = 16
_NW = _NC * _NS
_CH = 128
_CHG = 128
_SLABS = 6


def _ln(h, g, b):
    mu = jnp.mean(h, axis=-1, keepdims=True)
    var = jnp.mean((h - mu) ** 2, axis=-1, keepdims=True)
    return (h - mu) * lax.rsqrt(var + _EPS) * g + b


def _precomp_body(x_ref, w1a_ref, w1b_ref, b1_ref, p_ref, q_ref):
    xb = x_ref[...]
    p_ref[...] = (
        jnp.dot(xb, w1a_ref[...], preferred_element_type=jnp.float32) + b1_ref[...]
    )
    q_ref[...] = jnp.dot(xb, w1b_ref[...], preferred_element_type=jnp.float32)


def _edge_mlp_body(g_ref, a_ref, w1c_ref, g1_ref, be1_ref, w2_ref, b2_ref, o_ref):
    t = lax.dot_general(
        a_ref[...], w1c_ref[...], (((0,), (0,)), ((), ())),
        preferred_element_type=jnp.float32,
    )
    h = g_ref[...] + t
    h = jnp.maximum(_ln(h, g1_ref[...], be1_ref[...]), 0.0)
    m = (
        jnp.dot(
            h.astype(jnp.bfloat16), w2_ref[...], preferred_element_type=jnp.float32
        )
        + b2_ref[...]
    )
    o_ref[...] = jnp.maximum(m, 0.0)


def _update_body(
    x_ref, a0_ref, a1_ref, w3a_ref, w3b_ref, b3_ref, g3_ref, be3_ref, o_ref
):
    xb = x_ref[...]
    ag = a0_ref[...] + a1_ref[...]
    u = (
        jnp.dot(xb, w3a_ref[...], preferred_element_type=jnp.float32)
        + jnp.dot(ag, w3b_ref[...], preferred_element_type=jnp.float32)
        + b3_ref[...]
    )
    u = jnp.maximum(_ln(u, g3_ref[...], be3_ref[...]), 0.0)
    o_ref[...] = xb + u


def _make_warmup():
    mesh = plsc.VectorSubcoreMesh(core_axis_name="c", subcore_axis_name="s")

    @functools.partial(
        pl.kernel,
        mesh=mesh,
        out_type=jax.ShapeDtypeStruct((_NW, 128), jnp.int32),
        scratch_types=[pltpu.VMEM((128,), jnp.int32)],
    )
    def warm_k(in_hbm, out_hbm, buf):
        cid = lax.axis_index("c")
        sid = lax.axis_index("s")
        wid = sid * _NC + cid
        pltpu.sync_copy(in_hbm.at[wid], buf)
        pltpu.sync_copy(buf, out_hbm.at[wid])

    return warm_k


def _make_gather(d, e_pad):
    nch = e_pad // (_NW * _CHG)
    epw = nch * _CHG
    mesh = plsc.VectorSubcoreMesh(core_axis_name="c", subcore_axis_name="s")

    assert nch >= 4

    @functools.partial(
        pl.kernel,
        mesh=mesh,
        out_type=jax.ShapeDtypeStruct((e_pad, d), jnp.float32),
        scratch_types=[
            pltpu.VMEM((nch, _CHG), jnp.int32),
            pltpu.VMEM((nch, _CHG), jnp.int32),
            pltpu.VMEM((2, _CHG, d), jnp.float32),
            pltpu.VMEM((2, _CHG, d), jnp.float32),
            pltpu.SemaphoreType.DMA,
            pltpu.SemaphoreType.DMA,
            pltpu.SemaphoreType.DMA,
            pltpu.SemaphoreType.DMA,
        ],
    )
    def gather_k(
        p_hbm, q_hbm, row2_hbm, col2_hbm, out_hbm, ridx, cidx, bufp, bufq,
        sg0, sg1, sw0, sw1
    ):
        cid = lax.axis_index("c")
        sid = lax.axis_index("s")
        wid = sid * _NC + cid
        pltpu.sync_copy(row2_hbm.at[wid], ridx)
        pltpu.sync_copy(col2_hbm.at[wid], cidx)
        sg = (sg0, sg1)
        sw = (sw0, sw1)

        def issue(j, s):
            pltpu.async_copy(p_hbm.at[ridx.at[j]], bufp.at[s], sg[s])
            pltpu.async_copy(q_hbm.at[cidx.at[j]], bufq.at[s], sg[s])

        def wait_gather(s):
            pltpu.make_async_copy(p_hbm.at[ridx.at[0]], bufp.at[s], sg[s]).wait()
            pltpu.make_async_copy(q_hbm.at[cidx.at[0]], bufq.at[s], sg[s]).wait()

        def wait_write(s):
            pltpu.make_async_copy(
                bufp.at[s], out_hbm.at[pl.ds(wid * epw, _CHG)], sw[s]
            ).wait()

        def process(j, s):
            wait_gather(s)

            def addrow(r, c2):
                for k in range(d // 16):
                    v = bufq[s, r, pl.ds(k * 16, 16)]
                    plsc.addupdate(bufp.at[s, r, pl.ds(k * 16, 16)], v)
                return c2

            lax.fori_loop(0, _CHG, addrow, 0)
            pltpu.async_copy(
                bufp.at[s], out_hbm.at[pl.ds(wid * epw + j * _CHG, _CHG)], sw[s]
            )

        npairs = (nch - 2) // 2
        ntail = nch - 2 * npairs
        issue(0, 0)
        issue(1, 1)

        def pair(jj, carry):
            j = 2 * jj
            process(j, 0)
            wait_write(0)
            issue(j + 2, 0)
            process(j + 1, 1)
            wait_write(1)
            issue(j + 3, 1)
            return carry

        lax.fori_loop(0, npairs, pair, 0)
        t0 = 2 * npairs
        if ntail == 3:
            process(t0, 0)
            wait_write(0)
            issue(nch - 1, 0)
            process(t0 + 1, 1)
            process(nch - 1, 0)
        else:
            process(t0, 0)
            process(t0 + 1, 1)
        wait_write(0)
        wait_write(1)

    return gather_k


def _make_scatter(n_acc, d, slab_nchs, ntr, init):
    nslab = len(slab_nchs)
    nch_max = max(slab_nchs)
    mesh = plsc.VectorSubcoreMesh(core_axis_name="c", subcore_axis_name="s")

    assert min(slab_nchs) >= 4

    @functools.partial(
        pl.kernel,
        mesh=mesh,
        out_type=jax.ShapeDtypeStruct((_NC, n_acc, d), jnp.float32),
        scratch_types=[
            pltpu.VMEM((nch_max, _CH), jnp.int32),
            pltpu.VMEM((2, _CH, d), jnp.float32),
            pltpu.VMEM_SHARED((n_acc, d), jnp.float32),
            pltpu.SemaphoreType.DMA,
            pltpu.SemaphoreType.DMA,
        ],
    )
    def scatter_k(*refs):
        m_hbms = refs[0:nslab]
        r_hbms = refs[nslab : 2 * nslab]
        nin = 2 * nslab + (0 if init else 1)
        out_hbm = refs[nin]
        idx, buf, acc, sr0, sr1 = refs[nin + 1 :]
        sr = (sr0, sr1)
        cid = lax.axis_index("c")
        sid = lax.axis_index("s")
        wid = sid * _NC + cid
        z = jnp.zeros((16,), jnp.float32)

        if init:
            def zrow(i, carry):
                for k in range(d // 16):
                    buf[0, i, pl.ds(k * 16, 16)] = z
                return carry

            lax.fori_loop(0, _CH, zrow, 0)
            for off in range(0, ntr, _CH):
                sz = min(_CH, ntr - off)
                pltpu.sync_copy(
                    buf.at[0, pl.ds(0, sz)], acc.at[pl.ds(sid * ntr + off, sz)]
                )
        else:
            acc_in = refs[2 * nslab]
            for off in range(0, ntr, _CH):
                sz = min(_CH, ntr - off)
                pltpu.sync_copy(
                    acc_in.at[cid, pl.ds(sid * ntr + off, sz)],
                    buf.at[0, pl.ds(0, sz)],
                )
                pltpu.sync_copy(
                    buf.at[0, pl.ds(0, sz)], acc.at[pl.ds(sid * ntr + off, sz)]
                )
        plsc.subcore_barrier()

        for sb in range(nslab):
            m_hbm = m_hbms[sb]
            nch = slab_nchs[sb]
            epw = nch * _CH
            pltpu.sync_copy(r_hbms[sb].at[wid], idx.at[pl.ds(0, nch)])

            def issue(j, s):
                pltpu.async_copy(
                    m_hbm.at[pl.ds(wid * epw + j * _CH, _CH)], buf.at[s], sr[s]
                )

            def wait_read(s):
                pltpu.make_async_copy(
                    m_hbm.at[pl.ds(wid * epw, _CH)], buf.at[s], sr[s]
                ).wait()

            def process(j, s):
                wait_read(s)
                pltpu.sync_copy(buf.at[s], acc.at[idx.at[j]], add=True)

            npairs = (nch - 2) // 2
            ntail = nch - 2 * npairs
            issue(0, 0)
            issue(1, 1)

            def pair(jj, carry):
                j = 2 * jj
                process(j, 0)
                issue(j + 2, 0)
                process(j + 1, 1)
                issue(j + 3, 1)
                return carry

            lax.fori_loop(0, npairs, pair, 0)
            t0 = 2 * npairs
            if ntail == 3:
                process(t0, 0)
                issue(nch - 1, 0)
                process(t0 + 1, 1)
                process(nch - 1, 0)
            else:
                process(t0, 0)
                process(t0 + 1, 1)

        plsc.subcore_barrier()
        pltpu.sync_copy(
            acc.at[pl.ds(sid * ntr, ntr)], out_hbm.at[cid, pl.ds(sid * ntr, ntr)]
        )

    return scatter_k


def kernel(x, edge_index, edge_attr, W1, b1, g1, beta1, W2, b2, W3, b3, g3, beta3):
    n, d = x.shape
    e = edge_attr.shape[0]
    assert d % 16 == 0

    grain = _NW * _CH
    e_pad = -(-e // grain) * grain
    units = e_pad // grain
    fr = (0.05, 0.24, 0.24, 0.19, 0.15, 0.13)
    slab_units = [max(4, int(round(units * f))) for f in fr[:_SLABS]]
    slab_units[1] += units - sum(slab_units)

    ntr = -(-(n + 1) // _NS)
    ntr = -(-ntr // 8) * 8
    n_acc = ntr * _NS

    warm = _make_warmup()(jnp.zeros((_NW, 128), jnp.int32))
    wz = warm[0, 0]

    row = edge_index[0] + wz
    col = edge_index[1] + wz
    padg = jnp.zeros((e_pad - e,), jnp.int32)
    row_pg = jnp.concatenate([row, padg])
    col_pg = jnp.concatenate([col, padg])
    row_ps = jnp.concatenate([row, jnp.full((e_pad - e,), n, jnp.int32)])
    ea_t = jnp.pad(edge_attr.T, ((0, 0), (0, e_pad - e)))

    w1a = W1[:d]
    w1b = W1[d : 2 * d]
    w1c = W1[2 * d :]
    w3a = W3[:d]
    w3b = W3[d:]
    b1r = b1.reshape(1, d)
    g1r = g1.reshape(1, d)
    be1r = beta1.reshape(1, d)
    b2r = b2.reshape(1, d)
    b3r = b3.reshape(1, d)
    g3r = g3.reshape(1, d)
    be3r = beta3.reshape(1, d)

    bn = 1000 if n % 1000 == 0 else 8
    assert n % bn == 0
    full = pl.BlockSpec((d, d), lambda i: (0, 0))
    vec = pl.BlockSpec((1, d), lambda i: (0, 0))
    x_pad = jnp.pad(x, ((0, n_acc - n), (0, 0)))
    p_arr, q_arr = pl.pallas_call(
        _precomp_body,
        grid=(_NS,),
        in_specs=[pl.BlockSpec((ntr, d), lambda i: (i, 0)), full, full, vec],
        out_specs=[
            pl.BlockSpec((ntr, d), lambda i: (i, 0)),
            pl.BlockSpec((ntr, d), lambda i: (i, 0)),
        ],
        out_shape=[
            jax.ShapeDtypeStruct((n_acc, d), jnp.float32),
            jax.ShapeDtypeStruct((n_acc, d), jnp.float32),
        ],
    )(x_pad, w1a, w1b, b1r)

    be = 2048
    w2bf = W2.astype(jnp.bfloat16)
    gmakers = {}
    m_slabs = []
    rs_slabs = []
    offs = 0
    for s in range(_SLABS):
        e_sl = slab_units[s] * grain
        sl = slice(offs, offs + e_sl)
        offs += e_sl
        nch_g = e_sl // (_NW * _CHG)
        rg = row_pg[sl].reshape(_NW, nch_g, _CHG)
        cg = col_pg[sl].reshape(_NW, nch_g, _CHG)
        rs_slabs.append(row_ps[sl].reshape(_NW, slab_units[s], _CH))
        if e_sl not in gmakers:
            gmakers[e_sl] = _make_gather(d, e_sl)
        g_s = gmakers[e_sl](p_arr, q_arr, rg, cg)
        m_s = pl.pallas_call(
            _edge_mlp_body,
            grid=(e_sl // be,),
            in_specs=[
                pl.BlockSpec((be, d), lambda i: (i, 0)),
                pl.BlockSpec((2, be), lambda i: (0, i)),
                pl.BlockSpec((2, d), lambda i: (0, 0)),
                vec,
                vec,
                full,
                vec,
            ],
            out_specs=pl.BlockSpec((be, d), lambda i: (i, 0)),
            out_shape=jax.ShapeDtypeStruct((e_sl, d), jnp.float32),
        )(g_s, ea_t[:, sl], w1c, g1r, be1r, w2bf, b2r)
        m_slabs.append(m_s)

    ksp = _SLABS - 2
    parts_a = _make_scatter(n_acc, d, tuple(slab_units[:ksp]), ntr, True)(
        *m_slabs[:ksp], *rs_slabs[:ksp]
    )
    parts = _make_scatter(n_acc, d, tuple(slab_units[ksp:]), ntr, False)(
        *m_slabs[ksp:], *rs_slabs[ksp:], parts_a
    )
    a0 = parts[0, :n]
    a1 = parts[1, :n]

    out = pl.pallas_call(
        _update_body,
        grid=(n // bn,),
        in_specs=[
            pl.BlockSpec((bn, d), lambda i: (i, 0)),
            pl.BlockSpec((bn, d), lambda i: (i, 0)),
            pl.BlockSpec((bn, d), lambda i: (i, 0)),
            full,
            full,
            vec,
            vec,
            vec,
        ],
        out_specs=pl.BlockSpec((bn, d), lambda i: (i, 0)),
        out_shape=jax.ShapeDtypeStruct((n, d), jnp.float32),
    )(x, a0, a1, w3a, w3b, b3r, g3r, be3r)
    return out
